# Initial kernel scaffold; baseline (speedup 1.0000x reference)
import functools

import jax
import jax.numpy as jnp
from jax import lax
from jax.experimental import pallas as pl
from jax.experimental.pallas import tpu as pltpu
from jax.experimental.pallas import tpu_sc as plsc

_NCORES = 2
_NSUB = 16
_W = 32
_K = 160


@functools.lru_cache(maxsize=None)
def _make_sc_segsum(NR, D, NB):
  S = D // _W
  SPC = S // _NCORES
  RPT = NR // _NSUB
  ZR = 250
  mesh = plsc.VectorSubcoreMesh(
      core_axis_name="c", subcore_axis_name="s",
      num_cores=_NCORES, num_subcores=_NSUB)

  @functools.partial(
      pl.kernel,
      out_type=jax.ShapeDtypeStruct((NR, D), jnp.float32),
      mesh=mesh,
      scratch_types=[
          pltpu.VMEM_SHARED((NR, _W), jnp.float32),
          pltpu.VMEM((NB, _K), jnp.int32),
          pltpu.VMEM((NB, _K), jnp.int32),
          pltpu.VMEM((_K, _W), jnp.float32),
          pltpu.VMEM((_K, _W), jnp.float32),
          pltpu.VMEM((250, _W), jnp.float32),
          pltpu.SemaphoreType.DMA,
          pltpu.SemaphoreType.DMA,
      ],
  )
  def sc_segsum(h_hbm, src_hbm, dst_hbm, out_hbm,
                acc, sidx, didx, stg0, stg1, zbuf, sem0, sem1):
    cid = lax.axis_index("c")
    sid = lax.axis_index("s")

    @pl.loop(0, ZR)
    def _zero_zbuf(i):
      zbuf[i, pl.ds(0, 16)] = jnp.zeros((16,), jnp.float32)
      zbuf[i, pl.ds(16, 16)] = jnp.zeros((16,), jnp.float32)

    pltpu.sync_copy(src_hbm.at[sid], sidx)
    pltpu.sync_copy(dst_hbm.at[sid], didx)
    row0 = sid * RPT

    for core in range(_NCORES):

      @pl.when(cid == core)
      def _core_body():
        for si in range(SPC):
          c0 = (core * SPC + si) * _W

          for q in range(RPT // ZR):
            pltpu.sync_copy(zbuf, acc.at[pl.ds(row0 + q * ZR, ZR)])
          plsc.subcore_barrier()

          pltpu.async_copy(h_hbm.at[sidx.at[0], pl.ds(c0, _W)], stg0, sem0)
          pltpu.async_copy(h_hbm.at[sidx.at[1], pl.ds(c0, _W)], stg1, sem1)

          @pl.loop(0, NB, step=2)
          def _batches(b):
            for p, (stg, sem) in enumerate(((stg0, sem0), (stg1, sem1))):
              j = b + p
              pltpu.make_async_copy(
                  h_hbm.at[pl.ds(0, _K), pl.ds(c0, _W)], stg, sem).wait()
              pltpu.sync_copy(stg, acc.at[didx.at[j]], add=True)

              @pl.when(j + 2 < NB)
              def _next():
                pltpu.async_copy(
                    h_hbm.at[sidx.at[j + 2], pl.ds(c0, _W)], stg, sem)

          plsc.subcore_barrier()
          pltpu.sync_copy(
              acc.at[pl.ds(row0, RPT)],
              out_hbm.at[pl.ds(row0, RPT), pl.ds(c0, _W)])

  return sc_segsum


def _dot(a, b):
  return jnp.dot(a, b, preferred_element_type=jnp.float32)


def _accum_stats(stats_ref, y, first):
  @pl.when(first)
  def _():
    stats_ref[...] = jnp.zeros_like(stats_ref)

  part = jnp.stack([jnp.sum(y, axis=0), jnp.sum(y * y, axis=0)])
  stats_ref[...] += part


def _init_body(x_ref, mask_ref, fcw_ref, h_ref, c_ref, *, inv_r):
  r = pl.program_id(1)
  m = mask_ref[0, :]
  h = x_ref[...] * m[:, None]
  h_ref[...] = h

  @pl.when(r == 0)
  def _():
    c_ref[...] = jnp.zeros_like(c_ref)

  c_ref[...] += _dot(h, fcw_ref[...]) * inv_r


def _pass_a_body(h_ref, a_ref, w_ref, b_ref, y_ref, stats_ref):
  y = _dot(h_ref[...] + a_ref[...], w_ref[...]) + b_ref[...]
  y_ref[...] = y
  _accum_stats(stats_ref, y, pl.program_id(0) == 0)


def _norm(y, stats_ref, g_ref, b_ref, count):
  m = stats_ref[0, :] / count
  v = stats_ref[1, :] / count - m * m
  scale = g_ref[0, :] * lax.rsqrt(v + 1e-5)
  shift = b_ref[0, :] - m * scale
  return jnp.maximum(y * scale[None, :] + shift[None, :], 0.0)


def _pass_b_body(y1_ref, stats1_ref, g_ref, bb_ref, w_ref, b2_ref,
                 y2_ref, stats2_ref, *, count):
  z = _norm(y1_ref[...], stats1_ref, g_ref, bb_ref, count)
  y2 = _dot(z, w_ref[...]) + b2_ref[...]
  y2_ref[...] = y2
  _accum_stats(stats2_ref, y2, pl.program_id(0) == 0)


def _pass_c_body(y2_ref, stats_ref, g_ref, bb_ref, fcw_ref, h_ref, c_ref,
                 *, count, inv_r):
  r = pl.program_id(1)
  z = _norm(y2_ref[...], stats_ref, g_ref, bb_ref, count)
  h_ref[...] = z

  @pl.when(r == 0)
  def _():
    c_ref[...] = jnp.zeros_like(c_ref)

  c_ref[...] += _dot(z, fcw_ref[...]) * inv_r


def _final_body(batch_ref, bias_ref, *rest, nblk, G):
  *c_refs, out_ref = rest
  j = pl.program_id(0)
  csum = c_refs[0][...]
  for c in c_refs[1:]:
    csum = csum + c[...]
  seg = batch_ref[...]
  gidx = lax.broadcasted_iota(jnp.int32, (G, seg.shape[1]), 0)
  onehot = (gidx == seg).astype(jnp.float32)
  part = _dot(onehot, csum)

  @pl.when(j == 0)
  def _():
    out_ref[...] = jnp.zeros_like(out_ref)

  out_ref[...] += part

  @pl.when(j == nblk - 1)
  def _():
    o = out_ref[...] + bias_ref[...]
    mx = jnp.max(o, axis=1, keepdims=True)
    lse = jnp.log(jnp.sum(jnp.exp(o - mx), axis=1, keepdims=True)) + mx
    out_ref[...] = o - lse


def kernel(x, edge_index, batch, drop, params):
  N, NF = x.shape
  R = drop.shape[0]
  E = edge_index.shape[1]
  G = 64
  NC_OUT = params["fc0"]["w"].shape[1]
  D = params["conv0"]["w1"].shape[1]
  NR = R * N
  RE = R * E
  RB = 1000
  NBLK = NR // RB
  JBLK = N // RB
  NB = RE // (_NSUB * _K)
  f32 = jnp.float32

  off = (jnp.max(edge_index) + 1).astype(jnp.int32)
  r_off = jnp.arange(R, dtype=jnp.int32) * off
  src_rows = (edge_index[0][None, :] + r_off[:, None]).reshape(
      _NSUB, NB, _K)
  dst_rows = (edge_index[1][None, :] + r_off[:, None]).reshape(
      _NSUB, NB, _K)
  mask = 1.0 - drop.astype(f32)
  batch2d = batch.reshape(1, N)
  bias_sum = sum(params["fc%d" % i]["b"] for i in range(5)).reshape(1, NC_OUT)
  inv_r = 1.0 / R

  sc_segsum = _make_sc_segsum(NR, D, NB)

  row_spec = pl.BlockSpec((RB, D), lambda j, r: (r * JBLK + j, 0))
  const2 = lambda *_: (0, 0)

  h0, c0 = pl.pallas_call(
      functools.partial(_init_body, inv_r=inv_r),
      grid=(JBLK, R),
      in_specs=[
          pl.BlockSpec((RB, NF), lambda j, r: (j, 0)),
          pl.BlockSpec((1, RB), lambda j, r: (r, j)),
          pl.BlockSpec((NF, NC_OUT), const2),
      ],
      out_specs=[
          row_spec,
          pl.BlockSpec((RB, NC_OUT), lambda j, r: (j, 0)),
      ],
      out_shape=[
          jax.ShapeDtypeStruct((NR, D), f32),
          jax.ShapeDtypeStruct((N, NC_OUT), f32),
      ],
  )(x, mask, params["fc0"]["w"])

  h = h0
  cs = [c0]
  for i in range(4):
    p = params["conv%d" % i]
    ob = params["outer_bn%d" % i]
    agg = sc_segsum(h, src_rows, dst_rows)

    y1, stats1 = pl.pallas_call(
        _pass_a_body,
        grid=(NBLK,),
        in_specs=[
            pl.BlockSpec((RB, D), lambda j: (j, 0)),
            pl.BlockSpec((RB, D), lambda j: (j, 0)),
            pl.BlockSpec((D, D), lambda j: (0, 0)),
            pl.BlockSpec((1, D), lambda j: (0, 0)),
        ],
        out_specs=[
            pl.BlockSpec((RB, D), lambda j: (j, 0)),
            pl.BlockSpec((2, D), lambda j: (0, 0)),
        ],
        out_shape=[
            jax.ShapeDtypeStruct((NR, D), f32),
            jax.ShapeDtypeStruct((2, D), f32),
        ],
    )(h, agg, p["w1"], p["b1"].reshape(1, D))

    y2, stats2 = pl.pallas_call(
        functools.partial(_pass_b_body, count=float(NR)),
        grid=(NBLK,),
        in_specs=[
            pl.BlockSpec((RB, D), lambda j: (j, 0)),
            pl.BlockSpec((2, D), lambda j: (0, 0)),
            pl.BlockSpec((1, D), lambda j: (0, 0)),
            pl.BlockSpec((1, D), lambda j: (0, 0)),
            pl.BlockSpec((D, D), lambda j: (0, 0)),
            pl.BlockSpec((1, D), lambda j: (0, 0)),
        ],
        out_specs=[
            pl.BlockSpec((RB, D), lambda j: (j, 0)),
            pl.BlockSpec((2, D), lambda j: (0, 0)),
        ],
        out_shape=[
            jax.ShapeDtypeStruct((NR, D), f32),
            jax.ShapeDtypeStruct((2, D), f32),
        ],
    )(y1, stats1, p["bn_g"].reshape(1, D), p["bn_b"].reshape(1, D),
      p["w2"], p["b2"].reshape(1, D))

    h, ci = pl.pallas_call(
        functools.partial(_pass_c_body, count=float(NR), inv_r=inv_r),
        grid=(JBLK, R),
        in_specs=[
            row_spec,
            pl.BlockSpec((2, D), const2),
            pl.BlockSpec((1, D), const2),
            pl.BlockSpec((1, D), const2),
            pl.BlockSpec((D, NC_OUT), const2),
        ],
        out_specs=[
            row_spec,
            pl.BlockSpec((RB, NC_OUT), lambda j, r: (j, 0)),
        ],
        out_shape=[
            jax.ShapeDtypeStruct((NR, D), f32),
            jax.ShapeDtypeStruct((N, NC_OUT), f32),
        ],
    )(y2, stats2, ob["g"].reshape(1, D), ob["b"].reshape(1, D),
      params["fc%d" % (i + 1)]["w"])
    cs.append(ci)

  out = pl.pallas_call(
      functools.partial(_final_body, nblk=JBLK, G=G),
      grid=(JBLK,),
      in_specs=[
          pl.BlockSpec((1, RB), lambda j: (0, j)),
          pl.BlockSpec((1, NC_OUT), lambda j: (0, 0)),
      ] + [pl.BlockSpec((RB, NC_OUT), lambda j: (j, 0))] * 5,
      out_specs=pl.BlockSpec((G, NC_OUT), lambda j: (0, 0)),
      out_shape=jax.ShapeDtypeStruct((G, NC_OUT), f32),
  )(batch2d, bias_sum, *cs)

  return out


# speedup vs baseline: 2.0459x; 2.0459x over previous
import functools

import jax
import jax.numpy as jnp
from jax import lax
from jax.experimental import pallas as pl
from jax.experimental.pallas import tpu as pltpu
from jax.experimental.pallas import tpu_sc as plsc

_NCORES = 2
_NSUB = 16
_NT = _NCORES * _NSUB
_K = 128
_WV = 192


@functools.lru_cache(maxsize=None)
def _make_sc_segsum(NR, D, NBA, NWIN, NMETA):
  WPT = -(-NWIN // _NT)
  mesh = plsc.VectorSubcoreMesh(
      core_axis_name="c", subcore_axis_name="s",
      num_cores=_NCORES, num_subcores=_NSUB)

  @functools.partial(
      pl.kernel,
      out_type=jax.ShapeDtypeStruct((NWIN * _WV, D), jnp.float32),
      mesh=mesh,
      scratch_types=[
          pltpu.VMEM((_WV + 8, D), jnp.float32),
          pltpu.VMEM((2, _K), jnp.int32),
          pltpu.VMEM((_K, D), jnp.float32),
          pltpu.VMEM((NMETA,), jnp.int32),
          pltpu.SemaphoreType.DMA,
          pltpu.SemaphoreType.DMA,
      ],
  )
  def sc_segsum(h_hbm, idx_hbm, meta_hbm, out_hbm,
                acc, ib, stg, metab, si, sg):
    cid = lax.axis_index("c")
    sid = lax.axis_index("s")
    wid = cid * _NSUB + sid

    pltpu.sync_copy(meta_hbm, metab)

    for t in range(WPT):
      w = wid + t * _NT

      @pl.when(w < NWIN)
      def _window():
        @pl.loop(0, _WV + 8)
        def _zero(i):
          for c in range(D // 16):
            acc[i, pl.ds(c * 16, 16)] = jnp.zeros((16,), jnp.float32)

        mv = metab[pl.ds(w, 16)]
        b0 = mv[0]
        b1 = mv[1]

        @pl.loop(b0, b1)
        def _batch(b):
          pltpu.async_copy(idx_hbm.at[b], ib, si)
          pltpu.make_async_copy(idx_hbm.at[0], ib, si).wait()
          pltpu.async_copy(h_hbm.at[ib.at[0]], stg, sg)
          pltpu.make_async_copy(h_hbm.at[ib.at[0]], stg, sg).wait()

          @pl.loop(0, _K // 16)
          def _egroup(g):
            dvec = ib[1, pl.ds(g * 16, 16)]
            for i in range(16):
              d = dvec[i]
              e = g * 16 + i
              for c in range(D // 16):
                sl = pl.ds(c * 16, 16)
                plsc.addupdate(acc.at[d, sl], stg[e, sl])

        pltpu.sync_copy(
            acc.at[pl.ds(0, _WV)], out_hbm.at[pl.ds(w * _WV, _WV)])

  return sc_segsum


def _dot(a, b):
  return jnp.dot(a, b, preferred_element_type=jnp.float32)


def _accum_stats(stats_ref, y, first):
  @pl.when(first)
  def _():
    stats_ref[...] = jnp.zeros_like(stats_ref)

  part = jnp.stack([jnp.sum(y, axis=0), jnp.sum(y * y, axis=0)])
  stats_ref[...] += part


def _init_body(x_ref, mask_ref, fcw_ref, h_ref, c_ref, *, inv_r):
  r = pl.program_id(1)
  mb = mask_ref[...]
  sel = (lax.broadcasted_iota(jnp.int32, mb.shape, 1) == r).astype(mb.dtype)
  m = jnp.sum(mb * sel, axis=1)
  h = x_ref[...] * m[:, None]
  h_ref[...] = h

  @pl.when(r == 0)
  def _():
    c_ref[...] = jnp.zeros_like(c_ref)

  c_ref[...] += _dot(h, fcw_ref[...]) * inv_r


def _pass_a_body(h_ref, a_ref, w_ref, b_ref, y_ref, stats_ref):
  y = _dot(h_ref[...] + a_ref[...], w_ref[...]) + b_ref[...]
  y_ref[...] = y
  _accum_stats(stats_ref, y, pl.program_id(0) == 0)


def _norm(y, stats_ref, g_ref, b_ref, count):
  m = stats_ref[0, :] / count
  v = stats_ref[1, :] / count - m * m
  scale = g_ref[0, :] * lax.rsqrt(v + 1e-5)
  shift = b_ref[0, :] - m * scale
  return jnp.maximum(y * scale[None, :] + shift[None, :], 0.0)


def _pass_b_body(y1_ref, stats1_ref, g_ref, bb_ref, w_ref, b2_ref,
                 y2_ref, stats2_ref, *, count):
  z = _norm(y1_ref[...], stats1_ref, g_ref, bb_ref, count)
  y2 = _dot(z, w_ref[...]) + b2_ref[...]
  y2_ref[...] = y2
  _accum_stats(stats2_ref, y2, pl.program_id(0) == 0)


def _pass_c_body(y2_ref, stats_ref, g_ref, bb_ref, fcw_ref, h_ref, c_ref,
                 *, count, inv_r):
  r = pl.program_id(1)
  z = _norm(y2_ref[...], stats_ref, g_ref, bb_ref, count)
  h_ref[...] = z

  @pl.when(r == 0)
  def _():
    c_ref[...] = jnp.zeros_like(c_ref)

  c_ref[...] += _dot(z, fcw_ref[...]) * inv_r


def _final_body(batch_ref, bias_ref, *rest, nblk, G):
  *c_refs, out_ref = rest
  j = pl.program_id(0)
  csum = c_refs[0][...]
  for c in c_refs[1:]:
    csum = csum + c[...]
  seg = batch_ref[...]
  gidx = lax.broadcasted_iota(jnp.int32, (seg.shape[0], G), 1)
  onehot = (gidx == seg).astype(jnp.float32)
  part = lax.dot_general(
      onehot, csum, (((0,), (0,)), ((), ())),
      preferred_element_type=jnp.float32)

  @pl.when(j == 0)
  def _():
    out_ref[...] = jnp.zeros_like(out_ref)

  out_ref[...] += part

  @pl.when(j == nblk - 1)
  def _():
    o = out_ref[...] + bias_ref[...]
    mx = jnp.max(o, axis=1, keepdims=True)
    lse = jnp.log(jnp.sum(jnp.exp(o - mx), axis=1, keepdims=True)) + mx
    out_ref[...] = o - lse


def kernel(x, edge_index, batch, drop, params):
  N, NF = x.shape
  R = drop.shape[0]
  E = edge_index.shape[1]
  G = 64
  NC_OUT = params["fc0"]["w"].shape[1]
  D = params["conv0"]["w1"].shape[1]
  NR = R * N
  RE = R * E
  RB = 1000
  NBLK = NR // RB
  JBLK = N // RB
  NWIN = -(-NR // _WV)
  NBA = RE // _K + NWIN
  NMETA = -(-(NWIN + 16) // 8) * 8
  i32 = jnp.int32
  f32 = jnp.float32

  off = (jnp.max(edge_index) + 1).astype(i32)
  r_off = jnp.arange(R, dtype=i32) * off
  src_rows = (edge_index[0][None, :] + r_off[:, None]).reshape(-1)
  dst_rows = (edge_index[1][None, :] + r_off[:, None]).reshape(-1)
  sdst, ssrc = lax.sort([dst_rows, src_rows], num_keys=1)
  wbound = jnp.searchsorted(
      sdst, jnp.arange(NWIN + 1, dtype=i32) * _WV).astype(i32)
  lenw = wbound[1:] - wbound[:-1]
  pbo = jnp.concatenate(
      [jnp.zeros((1,), i32), jnp.cumsum(-(-lenw // _K)).astype(i32)])
  meta = jnp.pad(pbo, (0, NMETA - (NWIN + 1)))
  bidx = jnp.arange(NBA, dtype=i32)
  wob = jnp.clip(
      jnp.searchsorted(pbo, bidx, side="right").astype(i32) - 1,
      0, NWIN - 1)
  rel = (bidx - pbo[wob])[:, None] * _K + jnp.arange(_K, dtype=i32)[None, :]
  valid = rel < lenw[wob][:, None]
  i_e = jnp.clip(wbound[wob][:, None] + rel, 0, RE - 1)
  psrc = jnp.where(valid, ssrc[i_e], 0)
  dloc = jnp.where(valid, sdst[i_e] - (wob * _WV)[:, None], _WV)
  idx2 = jnp.stack([psrc, dloc], axis=1)
  mask = (1.0 - drop.astype(f32)).T
  batch2d = batch.reshape(N, 1)
  bias_sum = sum(params["fc%d" % i]["b"] for i in range(5)).reshape(1, NC_OUT)
  inv_r = 1.0 / R

  sc_segsum = _make_sc_segsum(NR, D, NBA, NWIN, NMETA)

  row_spec = pl.BlockSpec((RB, D), lambda j, r: (r * JBLK + j, 0))
  const2 = lambda *_: (0, 0)

  h0, c0 = pl.pallas_call(
      functools.partial(_init_body, inv_r=inv_r),
      grid=(JBLK, R),
      in_specs=[
          pl.BlockSpec((RB, NF), lambda j, r: (j, 0)),
          pl.BlockSpec((RB, R), lambda j, r: (j, 0)),
          pl.BlockSpec((NF, NC_OUT), const2),
      ],
      out_specs=[
          row_spec,
          pl.BlockSpec((RB, NC_OUT), lambda j, r: (j, 0)),
      ],
      out_shape=[
          jax.ShapeDtypeStruct((NR, D), f32),
          jax.ShapeDtypeStruct((N, NC_OUT), f32),
      ],
  )(x, mask, params["fc0"]["w"])

  h = h0
  cs = [c0]
  for i in range(4):
    p = params["conv%d" % i]
    ob = params["outer_bn%d" % i]
    agg = sc_segsum(h, idx2, meta)

    y1, stats1 = pl.pallas_call(
        _pass_a_body,
        grid=(NBLK,),
        in_specs=[
            pl.BlockSpec((RB, D), lambda j: (j, 0)),
            pl.BlockSpec((RB, D), lambda j: (j, 0)),
            pl.BlockSpec((D, D), lambda j: (0, 0)),
            pl.BlockSpec((1, D), lambda j: (0, 0)),
        ],
        out_specs=[
            pl.BlockSpec((RB, D), lambda j: (j, 0)),
            pl.BlockSpec((2, D), lambda j: (0, 0)),
        ],
        out_shape=[
            jax.ShapeDtypeStruct((NR, D), f32),
            jax.ShapeDtypeStruct((2, D), f32),
        ],
    )(h, agg, p["w1"], p["b1"].reshape(1, D))

    y2, stats2 = pl.pallas_call(
        functools.partial(_pass_b_body, count=float(NR)),
        grid=(NBLK,),
        in_specs=[
            pl.BlockSpec((RB, D), lambda j: (j, 0)),
            pl.BlockSpec((2, D), lambda j: (0, 0)),
            pl.BlockSpec((1, D), lambda j: (0, 0)),
            pl.BlockSpec((1, D), lambda j: (0, 0)),
            pl.BlockSpec((D, D), lambda j: (0, 0)),
            pl.BlockSpec((1, D), lambda j: (0, 0)),
        ],
        out_specs=[
            pl.BlockSpec((RB, D), lambda j: (j, 0)),
            pl.BlockSpec((2, D), lambda j: (0, 0)),
        ],
        out_shape=[
            jax.ShapeDtypeStruct((NR, D), f32),
            jax.ShapeDtypeStruct((2, D), f32),
        ],
    )(y1, stats1, p["bn_g"].reshape(1, D), p["bn_b"].reshape(1, D),
      p["w2"], p["b2"].reshape(1, D))

    h, ci = pl.pallas_call(
        functools.partial(_pass_c_body, count=float(NR), inv_r=inv_r),
        grid=(JBLK, R),
        in_specs=[
            row_spec,
            pl.BlockSpec((2, D), const2),
            pl.BlockSpec((1, D), const2),
            pl.BlockSpec((1, D), const2),
            pl.BlockSpec((D, NC_OUT), const2),
        ],
        out_specs=[
            row_spec,
            pl.BlockSpec((RB, NC_OUT), lambda j, r: (j, 0)),
        ],
        out_shape=[
            jax.ShapeDtypeStruct((NR, D), f32),
            jax.ShapeDtypeStruct((N, NC_OUT), f32),
        ],
    )(y2, stats2, ob["g"].reshape(1, D), ob["b"].reshape(1, D),
      params["fc%d" % (i + 1)]["w"])
    cs.append(ci)

  out = pl.pallas_call(
      functools.partial(_final_body, nblk=JBLK, G=G),
      grid=(JBLK,),
      in_specs=[
          pl.BlockSpec((RB, 1), lambda j: (j, 0)),
          pl.BlockSpec((1, NC_OUT), lambda j: (0, 0)),
      ] + [pl.BlockSpec((RB, NC_OUT), lambda j: (j, 0))] * 5,
      out_specs=pl.BlockSpec((G, NC_OUT), lambda j: (0, 0)),
      out_shape=jax.ShapeDtypeStruct((G, NC_OUT), f32),
  )(batch2d, bias_sum, *cs)

  return out


# speedup vs baseline: 2.3962x; 1.1713x over previous
import functools

import jax
import jax.numpy as jnp
from jax import lax
from jax.experimental import pallas as pl
from jax.experimental.pallas import tpu as pltpu
from jax.experimental.pallas import tpu_sc as plsc

_NCORES = 2
_NSUB = 16
_NT = _NCORES * _NSUB
_K = 128
_WV = 192


@functools.lru_cache(maxsize=None)
def _make_sc_segsum(NR, D, NBA, NWIN, NMETA):
  WPT = -(-NWIN // _NT)
  mesh = plsc.VectorSubcoreMesh(
      core_axis_name="c", subcore_axis_name="s",
      num_cores=_NCORES, num_subcores=_NSUB)

  @functools.partial(
      pl.kernel,
      out_type=jax.ShapeDtypeStruct((NWIN * _WV, D), jnp.float32),
      mesh=mesh,
      scratch_types=[
          pltpu.VMEM((_WV + 8, D), jnp.float32),
          pltpu.VMEM((2, _K), jnp.int32),
          pltpu.VMEM((2, _K), jnp.int32),
          pltpu.VMEM((_K,), jnp.int32),
          pltpu.VMEM((_K,), jnp.int32),
          pltpu.VMEM((_K, D), jnp.float32),
          pltpu.VMEM((_K, D), jnp.float32),
          pltpu.VMEM((NMETA,), jnp.int32),
          pltpu.SemaphoreType.DMA,
          pltpu.SemaphoreType.DMA,
          pltpu.SemaphoreType.DMA,
          pltpu.SemaphoreType.DMA,
      ],
  )
  def sc_segsum(h_hbm, idx_hbm, meta_hbm, out_hbm,
                acc, ib0, ib1, db0, db1, stg0, stg1, metab,
                si0, si1, sg0, sg1):
    cid = lax.axis_index("c")
    sid = lax.axis_index("s")
    wid = cid * _NSUB + sid
    ibs = (ib0, ib1)
    dbs = (db0, db1)
    sis = (si0, si1)
    stgs = (stg0, stg1)
    sgs = (sg0, sg1)

    pltpu.sync_copy(meta_hbm, metab)

    def issue_idx(b, p):
      pltpu.async_copy(idx_hbm.at[b], ibs[p], sis[p])

    def wait_idx(p):
      pltpu.make_async_copy(idx_hbm.at[0], ibs[p], sis[p]).wait()

    def issue_gather(p):
      pltpu.async_copy(h_hbm.at[ibs[p].at[0]], stgs[p], sgs[p])

    def wait_gather(p):
      pltpu.make_async_copy(h_hbm.at[ibs[p].at[0]], stgs[p], sgs[p]).wait()

    def snapshot(p):
      @pl.loop(0, _K // 16)
      def _snap(g):
        dbs[p][pl.ds(g * 16, 16)] = ibs[p][1, pl.ds(g * 16, 16)]

    def accumulate(p):
      @pl.loop(0, 4 * (_K // 16))
      def _egroup(g2):
        g = g2 // 4
        coff = (g2 % 4) * (D // 4)
        dvec = dbs[p][pl.ds(g * 16, 16)]
        for i in range(16):
          d = dvec[i]
          e = g * 16 + i
          for c in range(D // 64):
            sl = pl.ds(coff + c * 16, 16)
            plsc.addupdate(acc.at[d, sl], stgs[p][e, sl])

    for t in range(WPT):
      w = wid + t * _NT

      @pl.when(w < NWIN)
      def _window():
        @pl.loop(0, _WV + 8)
        def _zero(i):
          for c in range(D // 16):
            acc[i, pl.ds(c * 16, 16)] = jnp.zeros((16,), jnp.float32)

        mv = metab[pl.ds(w, 16)]
        b0 = mv[0]
        nb = mv[1] - mv[0]

        for j in range(2):
          @pl.when(nb > j)
          def _pro(j=j):
            issue_idx(b0 + j, j)
        for j in range(2):
          @pl.when(nb > j)
          def _pro2(j=j):
            wait_idx(j)
            issue_gather(j)

        @pl.loop(0, nb, step=2)
        def _batches(jj):
          for p in range(2):
            j = jj + p

            @pl.when(j < nb)
            def _one(j=j, p=p):
              wait_gather(p)
              snapshot(p)

              @pl.when(j + 2 < nb)
              def _refill(j=j, p=p):
                issue_idx(b0 + j + 2, p)

              accumulate(p)

              @pl.when(j + 2 < nb)
              def _next(p=p):
                wait_idx(p)
                issue_gather(p)

        pltpu.sync_copy(
            acc.at[pl.ds(0, _WV)], out_hbm.at[pl.ds(w * _WV, _WV)])

  return sc_segsum


def _dot(a, b):
  return jnp.dot(a, b, preferred_element_type=jnp.float32)


def _accum_stats(stats_ref, y, first):
  @pl.when(first)
  def _():
    stats_ref[...] = jnp.zeros_like(stats_ref)

  part = jnp.stack([jnp.sum(y, axis=0), jnp.sum(y * y, axis=0)])
  stats_ref[...] += part


def _init_body(x_ref, mask_ref, fcw_ref, h_ref, c_ref, *, inv_r):
  r = pl.program_id(1)
  mb = mask_ref[...]
  sel = (lax.broadcasted_iota(jnp.int32, mb.shape, 1) == r).astype(mb.dtype)
  m = jnp.sum(mb * sel, axis=1)
  h = x_ref[...] * m[:, None]
  h_ref[...] = h

  @pl.when(r == 0)
  def _():
    c_ref[...] = jnp.zeros_like(c_ref)

  c_ref[...] += _dot(h, fcw_ref[...]) * inv_r


def _pass_a_body(h_ref, a_ref, w_ref, b_ref, y_ref, stats_ref):
  y = _dot(h_ref[...] + a_ref[...], w_ref[...]) + b_ref[...]
  y_ref[...] = y
  _accum_stats(stats_ref, y, pl.program_id(0) == 0)


def _norm(y, stats_ref, g_ref, b_ref, count):
  m = stats_ref[0, :] / count
  v = stats_ref[1, :] / count - m * m
  scale = g_ref[0, :] * lax.rsqrt(v + 1e-5)
  shift = b_ref[0, :] - m * scale
  return jnp.maximum(y * scale[None, :] + shift[None, :], 0.0)


def _pass_b_body(y1_ref, stats1_ref, g_ref, bb_ref, w_ref, b2_ref,
                 y2_ref, stats2_ref, *, count):
  z = _norm(y1_ref[...], stats1_ref, g_ref, bb_ref, count)
  y2 = _dot(z, w_ref[...]) + b2_ref[...]
  y2_ref[...] = y2
  _accum_stats(stats2_ref, y2, pl.program_id(0) == 0)


def _pass_c_body(y2_ref, stats_ref, g_ref, bb_ref, fcw_ref, h_ref, c_ref,
                 *, count, inv_r):
  r = pl.program_id(1)
  z = _norm(y2_ref[...], stats_ref, g_ref, bb_ref, count)
  h_ref[...] = z

  @pl.when(r == 0)
  def _():
    c_ref[...] = jnp.zeros_like(c_ref)

  c_ref[...] += _dot(z, fcw_ref[...]) * inv_r


def _final_body(batch_ref, bias_ref, *rest, nblk, G):
  *c_refs, out_ref = rest
  j = pl.program_id(0)
  csum = c_refs[0][...]
  for c in c_refs[1:]:
    csum = csum + c[...]
  seg = batch_ref[...]
  gidx = lax.broadcasted_iota(jnp.int32, (seg.shape[0], G), 1)
  onehot = (gidx == seg).astype(jnp.float32)
  part = lax.dot_general(
      onehot, csum, (((0,), (0,)), ((), ())),
      preferred_element_type=jnp.float32)

  @pl.when(j == 0)
  def _():
    out_ref[...] = jnp.zeros_like(out_ref)

  out_ref[...] += part

  @pl.when(j == nblk - 1)
  def _():
    o = out_ref[...] + bias_ref[...]
    mx = jnp.max(o, axis=1, keepdims=True)
    lse = jnp.log(jnp.sum(jnp.exp(o - mx), axis=1, keepdims=True)) + mx
    out_ref[...] = o - lse


def kernel(x, edge_index, batch, drop, params):
  N, NF = x.shape
  R = drop.shape[0]
  E = edge_index.shape[1]
  G = 64
  NC_OUT = params["fc0"]["w"].shape[1]
  D = params["conv0"]["w1"].shape[1]
  NR = R * N
  RE = R * E
  RB = 1000
  NBLK = NR // RB
  JBLK = N // RB
  NWIN = -(-NR // _WV)
  NBA = RE // _K + NWIN
  NMETA = -(-(NWIN + 16) // 8) * 8
  i32 = jnp.int32
  f32 = jnp.float32

  off = (jnp.max(edge_index) + 1).astype(i32)
  r_off = jnp.arange(R, dtype=i32) * off
  src_rows = (edge_index[0][None, :] + r_off[:, None]).reshape(-1)
  dst_rows = (edge_index[1][None, :] + r_off[:, None]).reshape(-1)
  sdst, ssrc = lax.sort([dst_rows, src_rows], num_keys=1)
  wbound = jnp.searchsorted(
      sdst, jnp.arange(NWIN + 1, dtype=i32) * _WV).astype(i32)
  lenw = wbound[1:] - wbound[:-1]
  pbo = jnp.concatenate(
      [jnp.zeros((1,), i32), jnp.cumsum(-(-lenw // _K)).astype(i32)])
  meta = jnp.pad(pbo, (0, NMETA - (NWIN + 1)))
  bidx = jnp.arange(NBA, dtype=i32)
  wob = jnp.clip(
      jnp.searchsorted(pbo, bidx, side="right").astype(i32) - 1,
      0, NWIN - 1)
  rel = (bidx - pbo[wob])[:, None] * _K + jnp.arange(_K, dtype=i32)[None, :]
  valid = rel < lenw[wob][:, None]
  i_e = jnp.clip(wbound[wob][:, None] + rel, 0, RE - 1)
  psrc = jnp.where(valid, ssrc[i_e], 0)
  dloc = jnp.where(valid, sdst[i_e] - (wob * _WV)[:, None], _WV)
  idx2 = jnp.stack([psrc, dloc], axis=1)
  mask = (1.0 - drop.astype(f32)).T
  batch2d = batch.reshape(N, 1)
  bias_sum = sum(params["fc%d" % i]["b"] for i in range(5)).reshape(1, NC_OUT)
  inv_r = 1.0 / R

  sc_segsum = _make_sc_segsum(NR, D, NBA, NWIN, NMETA)

  row_spec = pl.BlockSpec((RB, D), lambda j, r: (r * JBLK + j, 0))
  const2 = lambda *_: (0, 0)

  h0, c0 = pl.pallas_call(
      functools.partial(_init_body, inv_r=inv_r),
      grid=(JBLK, R),
      in_specs=[
          pl.BlockSpec((RB, NF), lambda j, r: (j, 0)),
          pl.BlockSpec((RB, R), lambda j, r: (j, 0)),
          pl.BlockSpec((NF, NC_OUT), const2),
      ],
      out_specs=[
          row_spec,
          pl.BlockSpec((RB, NC_OUT), lambda j, r: (j, 0)),
      ],
      out_shape=[
          jax.ShapeDtypeStruct((NR, D), f32),
          jax.ShapeDtypeStruct((N, NC_OUT), f32),
      ],
  )(x, mask, params["fc0"]["w"])

  h = h0
  cs = [c0]
  for i in range(4):
    p = params["conv%d" % i]
    ob = params["outer_bn%d" % i]
    agg = sc_segsum(h, idx2, meta)

    y1, stats1 = pl.pallas_call(
        _pass_a_body,
        grid=(NBLK,),
        in_specs=[
            pl.BlockSpec((RB, D), lambda j: (j, 0)),
            pl.BlockSpec((RB, D), lambda j: (j, 0)),
            pl.BlockSpec((D, D), lambda j: (0, 0)),
            pl.BlockSpec((1, D), lambda j: (0, 0)),
        ],
        out_specs=[
            pl.BlockSpec((RB, D), lambda j: (j, 0)),
            pl.BlockSpec((2, D), lambda j: (0, 0)),
        ],
        out_shape=[
            jax.ShapeDtypeStruct((NR, D), f32),
            jax.ShapeDtypeStruct((2, D), f32),
        ],
    )(h, agg, p["w1"], p["b1"].reshape(1, D))

    y2, stats2 = pl.pallas_call(
        functools.partial(_pass_b_body, count=float(NR)),
        grid=(NBLK,),
        in_specs=[
            pl.BlockSpec((RB, D), lambda j: (j, 0)),
            pl.BlockSpec((2, D), lambda j: (0, 0)),
            pl.BlockSpec((1, D), lambda j: (0, 0)),
            pl.BlockSpec((1, D), lambda j: (0, 0)),
            pl.BlockSpec((D, D), lambda j: (0, 0)),
            pl.BlockSpec((1, D), lambda j: (0, 0)),
        ],
        out_specs=[
            pl.BlockSpec((RB, D), lambda j: (j, 0)),
            pl.BlockSpec((2, D), lambda j: (0, 0)),
        ],
        out_shape=[
            jax.ShapeDtypeStruct((NR, D), f32),
            jax.ShapeDtypeStruct((2, D), f32),
        ],
    )(y1, stats1, p["bn_g"].reshape(1, D), p["bn_b"].reshape(1, D),
      p["w2"], p["b2"].reshape(1, D))

    h, ci = pl.pallas_call(
        functools.partial(_pass_c_body, count=float(NR), inv_r=inv_r),
        grid=(JBLK, R),
        in_specs=[
            row_spec,
            pl.BlockSpec((2, D), const2),
            pl.BlockSpec((1, D), const2),
            pl.BlockSpec((1, D), const2),
            pl.BlockSpec((D, NC_OUT), const2),
        ],
        out_specs=[
            row_spec,
            pl.BlockSpec((RB, NC_OUT), lambda j, r: (j, 0)),
        ],
        out_shape=[
            jax.ShapeDtypeStruct((NR, D), f32),
            jax.ShapeDtypeStruct((N, NC_OUT), f32),
        ],
    )(y2, stats2, ob["g"].reshape(1, D), ob["b"].reshape(1, D),
      params["fc%d" % (i + 1)]["w"])
    cs.append(ci)

  out = pl.pallas_call(
      functools.partial(_final_body, nblk=JBLK, G=G),
      grid=(JBLK,),
      in_specs=[
          pl.BlockSpec((RB, 1), lambda j: (j, 0)),
          pl.BlockSpec((1, NC_OUT), lambda j: (0, 0)),
      ] + [pl.BlockSpec((RB, NC_OUT), lambda j: (j, 0))] * 5,
      out_specs=pl.BlockSpec((G, NC_OUT), lambda j: (0, 0)),
      out_shape=jax.ShapeDtypeStruct((G, NC_OUT), f32),
  )(batch2d, bias_sum, *cs)

  return out


# speedup vs baseline: 2.5557x; 1.0665x over previous
import functools

import jax
import jax.numpy as jnp
from jax import lax
from jax.experimental import pallas as pl
from jax.experimental.pallas import tpu as pltpu
from jax.experimental.pallas import tpu_sc as plsc

_NCORES = 2
_NSUB = 16
_NT = _NCORES * _NSUB
_K = 128
_WV = 192


@functools.lru_cache(maxsize=None)
def _make_sc_segsum(NR, D, NBA, NWIN, NMETA):
  WPT = -(-NWIN // _NT)
  mesh = plsc.VectorSubcoreMesh(
      core_axis_name="c", subcore_axis_name="s",
      num_cores=_NCORES, num_subcores=_NSUB)

  @functools.partial(
      pl.kernel,
      out_type=jax.ShapeDtypeStruct((NWIN * _WV, D), jnp.float32),
      mesh=mesh,
      scratch_types=[
          pltpu.VMEM((_WV + 8, D), jnp.float32),
          pltpu.VMEM((2, _K), jnp.int32),
          pltpu.VMEM((2, _K), jnp.int32),
          pltpu.VMEM((_K,), jnp.int32),
          pltpu.VMEM((_K,), jnp.int32),
          pltpu.VMEM((_K, D), jnp.float32),
          pltpu.VMEM((_K, D), jnp.float32),
          pltpu.VMEM((NMETA,), jnp.int32),
          pltpu.SemaphoreType.DMA,
          pltpu.SemaphoreType.DMA,
          pltpu.SemaphoreType.DMA,
          pltpu.SemaphoreType.DMA,
      ],
  )
  def sc_segsum(h_hbm, idx_hbm, meta_hbm, out_hbm,
                acc, ib0, ib1, db0, db1, stg0, stg1, metab,
                si0, si1, sg0, sg1):
    cid = lax.axis_index("c")
    sid = lax.axis_index("s")
    wid = cid * _NSUB + sid
    ibs = (ib0, ib1)
    dbs = (db0, db1)
    sis = (si0, si1)
    stgs = (stg0, stg1)
    sgs = (sg0, sg1)

    pltpu.sync_copy(meta_hbm, metab)

    def issue_idx(b, p):
      pltpu.async_copy(idx_hbm.at[b], ibs[p], sis[p])

    def wait_idx(p):
      pltpu.make_async_copy(idx_hbm.at[0], ibs[p], sis[p]).wait()

    def issue_gather(p):
      pltpu.async_copy(h_hbm.at[ibs[p].at[0]], stgs[p], sgs[p])

    def wait_gather(p):
      pltpu.make_async_copy(h_hbm.at[ibs[p].at[0]], stgs[p], sgs[p]).wait()

    def snapshot(p):
      @pl.loop(0, _K // 16)
      def _snap(g):
        dbs[p][pl.ds(g * 16, 16)] = ibs[p][1, pl.ds(g * 16, 16)]

    def accumulate(p):
      @pl.loop(0, 4 * (_K // 16))
      def _egroup(g2):
        g = g2 // 4
        coff = (g2 % 4) * (D // 4)
        dvec = dbs[p][pl.ds(g * 16, 16)]
        for i in range(16):
          d = dvec[i]
          e = g * 16 + i
          for c in range(D // 64):
            sl = pl.ds(coff + c * 16, 16)
            plsc.addupdate(acc.at[d, sl], stgs[p][e, sl])

    for t in range(WPT):
      w = wid + t * _NT

      @pl.when(w < NWIN)
      def _window():
        @pl.loop(0, _WV + 8)
        def _zero(i):
          for c in range(D // 16):
            acc[i, pl.ds(c * 16, 16)] = jnp.zeros((16,), jnp.float32)

        mv = metab[pl.ds(w, 16)]
        b0 = mv[0]
        nb = mv[1] - mv[0]

        for j in range(2):
          @pl.when(nb > j)
          def _pro(j=j):
            issue_idx(b0 + j, j)
        for j in range(2):
          @pl.when(nb > j)
          def _pro2(j=j):
            wait_idx(j)
            issue_gather(j)

        @pl.loop(0, nb, step=2)
        def _batches(jj):
          for p in range(2):
            j = jj + p

            @pl.when(j < nb)
            def _one(j=j, p=p):
              wait_gather(p)
              snapshot(p)

              @pl.when(j + 2 < nb)
              def _refill(j=j, p=p):
                issue_idx(b0 + j + 2, p)

              accumulate(p)

              @pl.when(j + 2 < nb)
              def _next(p=p):
                wait_idx(p)
                issue_gather(p)

        pltpu.sync_copy(
            acc.at[pl.ds(0, _WV)], out_hbm.at[pl.ds(w * _WV, _WV)])

  return sc_segsum


def _dot(a, b):
  return jnp.dot(a, b, preferred_element_type=jnp.float32)


def _accum_stats(stats_ref, y, first):
  @pl.when(first)
  def _():
    stats_ref[...] = jnp.zeros_like(stats_ref)

  part = jnp.stack([jnp.sum(y, axis=0), jnp.sum(y * y, axis=0)])
  stats_ref[...] += part


def _init_body(x_ref, mask_ref, fcw_ref, h_ref, c_ref, *, inv_r):
  r = pl.program_id(1)
  mb = mask_ref[...]
  sel = (lax.broadcasted_iota(jnp.int32, mb.shape, 1) == r).astype(mb.dtype)
  m = jnp.sum(mb * sel, axis=1)
  h = x_ref[...] * m[:, None]
  h_ref[...] = h

  @pl.when(r == 0)
  def _():
    c_ref[...] = jnp.zeros_like(c_ref)

  c_ref[...] += _dot(h, fcw_ref[...]) * inv_r


def _pass_a_body(h_ref, a_ref, w_ref, b_ref, y_ref, stats_ref):
  y = _dot(h_ref[...] + a_ref[...], w_ref[...]) + b_ref[...]
  y_ref[...] = y
  _accum_stats(stats_ref, y, pl.program_id(0) == 0)


def _norm(y, stats_ref, g_ref, b_ref, count):
  m = stats_ref[0, :] / count
  v = stats_ref[1, :] / count - m * m
  scale = g_ref[0, :] * lax.rsqrt(v + 1e-5)
  shift = b_ref[0, :] - m * scale
  return jnp.maximum(y * scale[None, :] + shift[None, :], 0.0)


def _pass_b_body(y1_ref, stats1_ref, g_ref, bb_ref, w_ref, b2_ref,
                 y2_ref, stats2_ref, *, count):
  z = _norm(y1_ref[...], stats1_ref, g_ref, bb_ref, count)
  y2 = _dot(z, w_ref[...]) + b2_ref[...]
  y2_ref[...] = y2
  _accum_stats(stats2_ref, y2, pl.program_id(0) == 0)


def _pass_c_body(y2_ref, stats_ref, g_ref, bb_ref, fcw_ref, h_ref, c_ref,
                 *, count, inv_r):
  r = pl.program_id(1)
  z = _norm(y2_ref[...], stats_ref, g_ref, bb_ref, count)
  h_ref[...] = z

  @pl.when(r == 0)
  def _():
    c_ref[...] = jnp.zeros_like(c_ref)

  c_ref[...] += _dot(z, fcw_ref[...]) * inv_r


def _final_body(batch_ref, bias_ref, *rest, nblk, G):
  *c_refs, out_ref = rest
  j = pl.program_id(0)
  csum = c_refs[0][...]
  for c in c_refs[1:]:
    csum = csum + c[...]
  seg = batch_ref[...]
  gidx = lax.broadcasted_iota(jnp.int32, (seg.shape[0], G), 1)
  onehot = (gidx == seg).astype(jnp.float32)
  part = lax.dot_general(
      onehot, csum, (((0,), (0,)), ((), ())),
      preferred_element_type=jnp.float32)

  @pl.when(j == 0)
  def _():
    out_ref[...] = jnp.zeros_like(out_ref)

  out_ref[...] += part

  @pl.when(j == nblk - 1)
  def _():
    o = out_ref[...] + bias_ref[...]
    mx = jnp.max(o, axis=1, keepdims=True)
    lse = jnp.log(jnp.sum(jnp.exp(o - mx), axis=1, keepdims=True)) + mx
    out_ref[...] = o - lse


def kernel(x, edge_index, batch, drop, params):
  N, NF = x.shape
  R = drop.shape[0]
  E = edge_index.shape[1]
  G = 64
  NC_OUT = params["fc0"]["w"].shape[1]
  D = params["conv0"]["w1"].shape[1]
  NR = R * N
  RE = R * E
  RB = 1000
  NBLK = NR // RB
  JBLK = N // RB
  NWIN = -(-NR // _WV)
  NBA = RE // _K + NWIN
  NMETA = -(-(NWIN + 16) // 8) * 8
  i32 = jnp.int32
  f32 = jnp.float32

  off = (jnp.max(edge_index) + 1).astype(i32)
  r_off = jnp.arange(R, dtype=i32) * off
  bdst, bsrc = lax.sort([edge_index[1], edge_index[0]], num_keys=1)
  sdst = (bdst[None, :] + r_off[:, None]).reshape(-1)
  ssrc = (bsrc[None, :] + r_off[:, None]).reshape(-1)
  wbound = jnp.searchsorted(
      sdst, jnp.arange(NWIN + 1, dtype=i32) * _WV).astype(i32)
  lenw = wbound[1:] - wbound[:-1]
  pbo = jnp.concatenate(
      [jnp.zeros((1,), i32), jnp.cumsum(-(-lenw // _K)).astype(i32)])
  meta = jnp.pad(pbo, (0, NMETA - (NWIN + 1)))
  bidx = jnp.arange(NBA, dtype=i32)
  wob = jnp.clip(
      jnp.searchsorted(pbo, bidx, side="right").astype(i32) - 1,
      0, NWIN - 1)
  rel = (bidx - pbo[wob])[:, None] * _K + jnp.arange(_K, dtype=i32)[None, :]
  valid = rel < lenw[wob][:, None]
  i_e = jnp.clip(wbound[wob][:, None] + rel, 0, RE - 1)
  psrc = jnp.where(valid, ssrc[i_e], 0)
  dloc = jnp.where(valid, sdst[i_e] - (wob * _WV)[:, None], _WV)
  idx2 = jnp.stack([psrc, dloc], axis=1)
  mask = (1.0 - drop.astype(f32)).T
  batch2d = batch.reshape(N, 1)
  bias_sum = sum(params["fc%d" % i]["b"] for i in range(5)).reshape(1, NC_OUT)
  inv_r = 1.0 / R

  sc_segsum = _make_sc_segsum(NR, D, NBA, NWIN, NMETA)

  row_spec = pl.BlockSpec((RB, D), lambda j, r: (r * JBLK + j, 0))
  const2 = lambda *_: (0, 0)

  h0, c0 = pl.pallas_call(
      functools.partial(_init_body, inv_r=inv_r),
      grid=(JBLK, R),
      in_specs=[
          pl.BlockSpec((RB, NF), lambda j, r: (j, 0)),
          pl.BlockSpec((RB, R), lambda j, r: (j, 0)),
          pl.BlockSpec((NF, NC_OUT), const2),
      ],
      out_specs=[
          row_spec,
          pl.BlockSpec((RB, NC_OUT), lambda j, r: (j, 0)),
      ],
      out_shape=[
          jax.ShapeDtypeStruct((NR, D), f32),
          jax.ShapeDtypeStruct((N, NC_OUT), f32),
      ],
  )(x, mask, params["fc0"]["w"])

  h = h0
  cs = [c0]
  for i in range(4):
    p = params["conv%d" % i]
    ob = params["outer_bn%d" % i]
    agg = sc_segsum(h, idx2, meta)

    y1, stats1 = pl.pallas_call(
        _pass_a_body,
        grid=(NBLK,),
        in_specs=[
            pl.BlockSpec((RB, D), lambda j: (j, 0)),
            pl.BlockSpec((RB, D), lambda j: (j, 0)),
            pl.BlockSpec((D, D), lambda j: (0, 0)),
            pl.BlockSpec((1, D), lambda j: (0, 0)),
        ],
        out_specs=[
            pl.BlockSpec((RB, D), lambda j: (j, 0)),
            pl.BlockSpec((2, D), lambda j: (0, 0)),
        ],
        out_shape=[
            jax.ShapeDtypeStruct((NR, D), f32),
            jax.ShapeDtypeStruct((2, D), f32),
        ],
    )(h, agg, p["w1"], p["b1"].reshape(1, D))

    y2, stats2 = pl.pallas_call(
        functools.partial(_pass_b_body, count=float(NR)),
        grid=(NBLK,),
        in_specs=[
            pl.BlockSpec((RB, D), lambda j: (j, 0)),
            pl.BlockSpec((2, D), lambda j: (0, 0)),
            pl.BlockSpec((1, D), lambda j: (0, 0)),
            pl.BlockSpec((1, D), lambda j: (0, 0)),
            pl.BlockSpec((D, D), lambda j: (0, 0)),
            pl.BlockSpec((1, D), lambda j: (0, 0)),
        ],
        out_specs=[
            pl.BlockSpec((RB, D), lambda j: (j, 0)),
            pl.BlockSpec((2, D), lambda j: (0, 0)),
        ],
        out_shape=[
            jax.ShapeDtypeStruct((NR, D), f32),
            jax.ShapeDtypeStruct((2, D), f32),
        ],
    )(y1, stats1, p["bn_g"].reshape(1, D), p["bn_b"].reshape(1, D),
      p["w2"], p["b2"].reshape(1, D))

    h, ci = pl.pallas_call(
        functools.partial(_pass_c_body, count=float(NR), inv_r=inv_r),
        grid=(JBLK, R),
        in_specs=[
            row_spec,
            pl.BlockSpec((2, D), const2),
            pl.BlockSpec((1, D), const2),
            pl.BlockSpec((1, D), const2),
            pl.BlockSpec((D, NC_OUT), const2),
        ],
        out_specs=[
            row_spec,
            pl.BlockSpec((RB, NC_OUT), lambda j, r: (j, 0)),
        ],
        out_shape=[
            jax.ShapeDtypeStruct((NR, D), f32),
            jax.ShapeDtypeStruct((N, NC_OUT), f32),
        ],
    )(y2, stats2, ob["g"].reshape(1, D), ob["b"].reshape(1, D),
      params["fc%d" % (i + 1)]["w"])
    cs.append(ci)

  out = pl.pallas_call(
      functools.partial(_final_body, nblk=JBLK, G=G),
      grid=(JBLK,),
      in_specs=[
          pl.BlockSpec((RB, 1), lambda j: (j, 0)),
          pl.BlockSpec((1, NC_OUT), lambda j: (0, 0)),
      ] + [pl.BlockSpec((RB, NC_OUT), lambda j: (j, 0))] * 5,
      out_specs=pl.BlockSpec((G, NC_OUT), lambda j: (0, 0)),
      out_shape=jax.ShapeDtypeStruct((G, NC_OUT), f32),
  )(batch2d, bias_sum, *cs)

  return out
